# Initial kernel scaffold; baseline (speedup 1.0000x reference)
import jax
import jax.numpy as jnp
from jax.experimental import pallas as pl


def kernel(feat_image, feat_gene, feat_text, src_i2i, dst_i2i, sim_i2i, src_g2i, dst_g2i, sim_g2i, src_t2i, dst_t2i, sim_t2i, src_i2g, dst_i2g, sim_i2g, src_i2t, dst_i2t, sim_i2t, Wad, bad, Wk, bk, Wq, bq, Wv, bv, Wa, ba, ew, eb, skip, Wlp, blp, Wattn, Whead1, bhead1, Whead, bhead):
    raise NotImplementedError("write your pallas kernel here")


# speedup vs baseline: 16.2813x; 16.2813x over previous
import functools

import jax
import jax.numpy as jnp
from jax import lax
from jax.experimental import pallas as pl
from jax.experimental.pallas import tpu as pltpu
from jax.experimental.pallas import tpu_sc as plsc

NODE_TYPES = ("image", "gene", "text")
TYPE_IX = {"image": 0, "gene": 1, "text": 2}
EDGE_TYPES = (
    ("image", "image", "i2i"),
    ("gene", "image", "g2i"),
    ("text", "image", "t2i"),
    ("image", "gene", "i2g"),
    ("image", "text", "i2t"),
)
HID = 256
NH = 8
DK = 32
SQRT_DK = float(DK) ** 0.5
LANES = 16
NC, NS, NW = 2, 16, 32
B_EDGE = 64


def _ceil_to(x, m):
    return (x + m - 1) // m * m


def _mm_body(x_ref, w_ref, b_ref, o_ref):
    o_ref[...] = (
        jnp.dot(x_ref[...], w_ref[...], preferred_element_type=jnp.float32)
        + b_ref[...]
    )


def _matmul_bias(x, w, b, bn=512):
    n, kdim = x.shape
    m = w.shape[1]
    npad = _ceil_to(n, bn)
    if npad != n:
        x = jnp.pad(x, ((0, npad - n), (0, 0)))
    out = pl.pallas_call(
        _mm_body,
        grid=(npad // bn,),
        in_specs=[
            pl.BlockSpec((bn, kdim), lambda i: (i, 0)),
            pl.BlockSpec((kdim, m), lambda i: (0, 0)),
            pl.BlockSpec((1, m), lambda i: (0, 0)),
        ],
        out_specs=pl.BlockSpec((bn, m), lambda i: (i, 0)),
        out_shape=jax.ShapeDtypeStruct((npad, m), jnp.float32),
    )(x, w, b.reshape(1, m))
    return out[:n]


def _make_newh_body(n_et):
    def body(*refs):
        h_ref = refs[0]
        p_refs = refs[1:1 + 6 * n_et]
        e8h = refs[1 + 6 * n_et][...]
        wa_top = refs[2 + 6 * n_et][...]
        wa_bot = refs[3 + 6 * n_et][...]
        ba = refs[4 + 6 * n_et][...]
        alpha = refs[5 + 6 * n_et][0, 0]
        o_ref = refs[6 + 6 * n_et]
        tsum_e = tsum_o = None
        for t in range(n_et):
            ae = p_refs[6 * t][...] + p_refs[6 * t + 1][...]
            ao = p_refs[6 * t + 2][...] + p_refs[6 * t + 3][...]
            dd = p_refs[6 * t + 4][...] + p_refs[6 * t + 5][...]
            denf = jnp.maximum(
                jnp.dot(dd, e8h, preferred_element_type=jnp.float32), 1e-30)
            if tsum_e is None:
                tsum_e, tsum_o = ae / denf, ao / denf
            else:
                tsum_e, tsum_o = tsum_e + ae / denf, tsum_o + ao / denf
        trans = (
            jnp.dot(tsum_e * (1.0 / n_et), wa_top,
                    preferred_element_type=jnp.float32)
            + jnp.dot(tsum_o * (1.0 / n_et), wa_bot,
                      preferred_element_type=jnp.float32)
            + ba
        )
        o_ref[...] = trans * alpha + h_ref[...] * (1.0 - alpha)
    return body


def _new_h(h, plist, e8h, wa_perm, ba, alpha, bn=512):
    n = h.shape[0]
    npad = _ceil_to(n, bn)
    n_et = len(plist)
    hp = jnp.pad(h, ((0, npad - n), (0, 0))) if npad != n else h
    args = [hp]
    in_specs = [pl.BlockSpec((bn, HID), lambda i: (i, 0))]
    for ae, ao, den in plist:
        for part, width in ((ae, 128), (ao, 128), (den, NH)):
            for ci in range(NC):
                pc = part[ci]
                if npad != n:
                    pc = jnp.pad(pc, ((0, npad - n), (0, 0)))
                args.append(pc)
                in_specs.append(pl.BlockSpec((bn, width), lambda i: (i, 0)))
    args += [e8h, wa_perm[:128], wa_perm[128:],
             ba.reshape(1, HID), alpha.reshape(1, 1)]
    in_specs += [
        pl.BlockSpec((NH, 128), lambda i: (0, 0)),
        pl.BlockSpec((128, HID), lambda i: (0, 0)),
        pl.BlockSpec((128, HID), lambda i: (0, 0)),
        pl.BlockSpec((1, HID), lambda i: (0, 0)),
        pl.BlockSpec((1, 1), lambda i: (0, 0)),
    ]
    out = pl.pallas_call(
        _make_newh_body(n_et),
        grid=(npad // bn,),
        in_specs=in_specs,
        out_specs=pl.BlockSpec((bn, HID), lambda i: (i, 0)),
        out_shape=jax.ShapeDtypeStruct((npad, HID), jnp.float32),
    )(*args)
    return out[:n]


def _head_body(h_ref, wlp_ref, blp_ref, w1_ref, b1_ref, wh_ref, bh_ref, o_ref):
    pooled = jnp.mean(h_ref[...], axis=0, keepdims=True)
    oh = jnp.dot(pooled, wlp_ref[...], preferred_element_type=jnp.float32) + blp_ref[...]
    g = jnp.dot(oh, w1_ref[...], preferred_element_type=jnp.float32) + b1_ref[...]
    o_ref[...] = jnp.dot(g, wh_ref[...], preferred_element_type=jnp.float32) + bh_ref[...]


def _head_rowbias(h_img, wlp, blp, w1, b1, wh, bh):
    n = h_img.shape[0]
    return pl.pallas_call(
        _head_body,
        in_specs=[
            pl.BlockSpec((n, HID), lambda: (0, 0)),
            pl.BlockSpec((HID, HID), lambda: (0, 0)),
            pl.BlockSpec((1, HID), lambda: (0, 0)),
            pl.BlockSpec((HID, 512), lambda: (0, 0)),
            pl.BlockSpec((1, 512), lambda: (0, 0)),
            pl.BlockSpec((512, HID), lambda: (0, 0)),
            pl.BlockSpec((1, HID), lambda: (0, 0)),
        ],
        out_specs=pl.BlockSpec((1, HID), lambda: (0, 0)),
        out_shape=jax.ShapeDtypeStruct((1, HID), jnp.float32),
    )(h_img, wlp, blp.reshape(1, HID), w1, b1.reshape(1, 512), wh, bh.reshape(1, HID))


def _edge_pass(ke, ko, qe, qo, ve, vo, src, dst, easc, nd):
    e = src.shape[0]
    e_pad = _ceil_to(e, NW * B_EDGE)
    pad = e_pad - e
    if pad:
        src = jnp.concatenate([src, jnp.zeros((pad,), jnp.int32)])
        dst = jnp.concatenate([dst, jnp.full((pad,), nd, jnp.int32)])
        easc = jnp.concatenate([easc, jnp.zeros((pad,), jnp.float32)])
    dstrow = lax.shift_right_logical(dst, 4)
    nd_pad = _ceil_to(nd + 1, 2048)
    nrows_den = nd_pad // 16
    epw = e_pad // NW
    nchunks = epw // B_EDGE
    rows_z = nd_pad // NS
    rows_dz = nrows_den // NS
    nhalf = 128

    def body(ke_hbm, ko_hbm, qe_hbm, qo_hbm, ve_hbm, vo_hbm,
             src_hbm, dst_hbm, dstrow_hbm, easc_hbm, z_hbm,
             agge_hbm, aggo_hbm, den_hbm,
             src_v, dst_v, dst_w, dstrow_v, easc_v,
             b1, b2, accbuf, exbuf, denrow,
             agge_s, aggo_s, den_s, sem1, sem2):
        c = lax.axis_index("c")
        s = lax.axis_index("s")
        wid = s * NC + c
        zero16 = jnp.zeros((LANES,), jnp.float32)
        iota16 = lax.iota(jnp.int32, LANES)
        perm = iota16 ^ 8
        lmask = iota16 < NH
        iota8m = iota16 & 7
        zr0 = s * rows_z
        dzr0 = s * rows_dz
        pltpu.sync_copy(z_hbm.at[pl.ds(zr0, rows_z)],
                        agge_s.at[pl.ds(zr0, rows_z)])
        pltpu.sync_copy(z_hbm.at[pl.ds(zr0, rows_z)],
                        aggo_s.at[pl.ds(zr0, rows_z)])
        pltpu.sync_copy(z_hbm.at[pl.ds(dzr0, rows_dz)],
                        den_s.at[pl.ds(dzr0, rows_dz)])
        plsc.subcore_barrier()

        ebase = wid * epw
        ngroups = B_EDGE // LANES

        def chunk(i, _):
            b0 = ebase + i * B_EDGE
            pltpu.sync_copy(src_hbm.at[pl.ds(b0, B_EDGE)], src_v)
            pltpu.sync_copy(dst_hbm.at[pl.ds(b0, B_EDGE)], dst_v)
            pltpu.sync_copy(dst_hbm.at[pl.ds(b0, B_EDGE)], dst_w)
            pltpu.sync_copy(dstrow_hbm.at[pl.ds(b0, B_EDGE)], dstrow_v)
            pltpu.sync_copy(easc_hbm.at[pl.ds(b0, B_EDGE)], easc_v)

            cp1 = pltpu.async_copy(ke_hbm.at[src_v], b1, sem1)
            cp2 = pltpu.async_copy(qe_hbm.at[dst_v], b2, sem2)
            cp1.wait()
            cp2.wait()

            def group1(g, _):
                for j in range(LANES):
                    ej = g * LANES + j
                    acc = zero16
                    for r in range(nhalf // LANES):
                        acc = acc + (b1[ej, pl.ds(r * LANES, LANES)]
                                     * b2[ej, pl.ds(r * LANES, LANES)])
                    accbuf[pl.ds(ej * LANES, LANES)] = acc
                return 0

            lax.fori_loop(0, ngroups, group1, 0)

            cp1 = pltpu.async_copy(ko_hbm.at[src_v], b1, sem1)
            cp2 = pltpu.async_copy(qo_hbm.at[dst_v], b2, sem2)
            cp1.wait()
            cp2.wait()

            def group2(g, _):
                e0 = g * LANES
                dst16 = dst_w[pl.ds(e0, LANES)]
                easc16 = easc_v[pl.ds(e0, LANES)]
                for j in range(LANES):
                    ej = e0 + j
                    acc = accbuf[pl.ds(ej * LANES, LANES)]
                    for r in range(nhalf // LANES):
                        acc = acc + (b1[ej, pl.ds(r * LANES, LANES)]
                                     * b2[ej, pl.ds(r * LANES, LANES)])
                    accs = lax.gather(
                        acc, perm[:, None],
                        lax.GatherDimensionNumbers(
                            offset_dims=(), collapsed_slice_dims=(0,),
                            start_index_map=(0,)),
                        slice_sizes=(1,),
                        mode=lax.GatherScatterMode.PROMISE_IN_BOUNDS)
                    ex = jnp.exp((acc + accs)
                                 * jnp.full((LANES,), easc16[j]))
                    exbuf[pl.ds(ej * LANES, LANES)] = ex
                    for q in range(nhalf // LANES):
                        denrow[ej, pl.ds(q * LANES, LANES)] = zero16
                    col = (dst16[j] & 15) * NH + iota8m
                    plsc.store_scatter(
                        denrow, [jnp.full((LANES,), ej, jnp.int32), col],
                        ex, mask=lmask)
                return 0

            lax.fori_loop(0, ngroups, group2, 0)

            for half_hbm, agg_s in ((ve_hbm, agge_s), (vo_hbm, aggo_s)):
                cp1 = pltpu.async_copy(half_hbm.at[src_v], b1, sem1)
                cp1.wait()

                def group3(g, _):
                    for j in range(LANES):
                        ej = g * LANES + j
                        ex = exbuf[pl.ds(ej * LANES, LANES)]
                        for r in range(nhalf // LANES):
                            b1[ej, pl.ds(r * LANES, LANES)] = (
                                b1[ej, pl.ds(r * LANES, LANES)] * ex)
                    return 0

                lax.fori_loop(0, ngroups, group3, 0)
                pltpu.sync_copy(b1, agg_s.at[dst_v], add=True)
            pltpu.sync_copy(denrow, den_s.at[dstrow_v], add=True)
            return 0

        lax.fori_loop(0, nchunks, chunk, 0)
        plsc.subcore_barrier()
        pltpu.sync_copy(agge_s.at[pl.ds(zr0, rows_z)],
                        agge_hbm.at[c, pl.ds(zr0, rows_z)])
        pltpu.sync_copy(aggo_s.at[pl.ds(zr0, rows_z)],
                        aggo_hbm.at[c, pl.ds(zr0, rows_z)])
        pltpu.sync_copy(den_s.at[pl.ds(dzr0, rows_dz)],
                        den_hbm.at[c, pl.ds(dzr0, rows_dz)])

    mesh = plsc.VectorSubcoreMesh(core_axis_name="c", subcore_axis_name="s")
    f = pl.kernel(
        body,
        mesh=mesh,
        compiler_params=pltpu.CompilerParams(needs_layout_passes=False),
        out_type=(
            jax.ShapeDtypeStruct((NC, nd_pad, 128), jnp.float32),
            jax.ShapeDtypeStruct((NC, nd_pad, 128), jnp.float32),
            jax.ShapeDtypeStruct((NC, nrows_den, 128), jnp.float32),
        ),
        scratch_types=[
            pltpu.VMEM((B_EDGE,), jnp.int32),
            pltpu.VMEM((B_EDGE,), jnp.int32),
            pltpu.VMEM((B_EDGE,), jnp.int32),
            pltpu.VMEM((B_EDGE,), jnp.int32),
            pltpu.VMEM((B_EDGE,), jnp.float32),
            pltpu.VMEM((B_EDGE, 128), jnp.float32),
            pltpu.VMEM((B_EDGE, 128), jnp.float32),
            pltpu.VMEM((B_EDGE * LANES,), jnp.float32),
            pltpu.VMEM((B_EDGE * LANES,), jnp.float32),
            pltpu.VMEM((B_EDGE, 128), jnp.float32),
            pltpu.VMEM_SHARED((nd_pad, 128), jnp.float32),
            pltpu.VMEM_SHARED((nd_pad, 128), jnp.float32),
            pltpu.VMEM_SHARED((nrows_den, 128), jnp.float32),
            pltpu.SemaphoreType.DMA,
            pltpu.SemaphoreType.DMA,
        ],
    )
    zeros = jnp.zeros((nd_pad, 128), jnp.float32)
    agg_e, agg_o, den = f(ke, ko, qe, qo, ve, vo, src, dst, dstrow, easc,
                          zeros)
    den = den.reshape(NC, nd_pad, NH)
    return agg_e[:, :nd], agg_o[:, :nd], den[:, :nd]


def _perm_cols(w):
    return w.reshape(w.shape[:-1] + (NH, DK)).swapaxes(-2, -1).reshape(w.shape)


def _perm_rows(w):
    return (w.reshape((NH, DK) + w.shape[-1:])
            .swapaxes(0, 1).reshape((NH * DK,) + w.shape[-1:]))


def kernel(feat_image, feat_gene, feat_text, src_i2i, dst_i2i, sim_i2i,
           src_g2i, dst_g2i, sim_g2i, src_t2i, dst_t2i, sim_t2i,
           src_i2g, dst_i2g, sim_i2g, src_i2t, dst_i2t, sim_i2t,
           Wad, bad, Wk, bk, Wq, bq, Wv, bv, Wa, ba, ew, eb, skip,
           Wlp, blp, Wattn, Whead1, bhead1, Whead, bhead):
    feats = {"image": feat_image, "gene": feat_gene, "text": feat_text}
    srcs = {"i2i": src_i2i, "g2i": src_g2i, "t2i": src_t2i,
            "i2g": src_i2g, "i2t": src_i2t}
    dsts = {"i2i": dst_i2i, "g2i": dst_g2i, "t2i": dst_t2i,
            "i2g": dst_i2g, "i2t": dst_i2t}
    sims = {"i2i": sim_i2i, "g2i": sim_g2i, "t2i": sim_t2i,
            "i2g": sim_i2g, "i2t": sim_i2t}
    nnodes = {nt: feats[nt].shape[0] for nt in NODE_TYPES}

    li = jnp.arange(NH)[:, None]
    ji = jnp.arange(128)[None, :]
    e8h = (li == (ji % NH)).astype(jnp.float32)

    h = {nt: _matmul_bias(feats[nt], Wad[TYPE_IX[nt]], bad[TYPE_IX[nt]])
         for nt in NODE_TYPES}

    for l in range(2):
        etypes_l = [et for et in EDGE_TYPES if l == 0 or et[1] == "image"]
        src_types = {st for st, _, _ in etypes_l}
        dst_types = {dt for _, dt, _ in etypes_l}
        kqv = {}
        for nt in NODE_TYPES:
            if nt not in src_types and nt not in dst_types:
                continue
            i = TYPE_IX[nt]
            w3 = jnp.concatenate(
                [_perm_cols(Wk[l, i]), _perm_cols(Wq[l, i]),
                 _perm_cols(Wv[l, i])], axis=1)
            b3 = jnp.concatenate(
                [_perm_cols(bk[l, i]), _perm_cols(bq[l, i]),
                 _perm_cols(bv[l, i])], axis=0)
            proj = _matmul_bias(h[nt], w3, b3)
            kqv[nt] = tuple(proj[:, 128 * t:128 * (t + 1)] for t in range(6))
        partials = {}
        for st, dt, en in etypes_l:
            easc = (sims[en] * ew[l] + eb[l]) * (1.0 / SQRT_DK)
            partials[en] = _edge_pass(
                kqv[st][0], kqv[st][1], kqv[dt][2], kqv[dt][3],
                kqv[st][4], kqv[st][5],
                srcs[en], dsts[en], easc, nnodes[dt])
        new_h = dict(h)
        for dt in NODE_TYPES:
            plist = [partials[en] for st, dtt, en in etypes_l if dtt == dt]
            if not plist:
                continue
            i = TYPE_IX[dt]
            alpha = jax.nn.sigmoid(skip[l, i])
            new_h[dt] = _new_h(h[dt], plist, e8h, _perm_rows(Wa[l, i]),
                               ba[l, i], alpha)
        h = new_h

    rowbias = _head_rowbias(h["image"], Wlp[0], blp[0],
                            Whead1, bhead1, Whead, bhead)
    return _matmul_bias(feat_image, Whead, rowbias)


# speedup vs baseline: 19.4642x; 1.1955x over previous
import functools

import jax
import jax.numpy as jnp
from jax import lax
from jax.experimental import pallas as pl
from jax.experimental.pallas import tpu as pltpu
from jax.experimental.pallas import tpu_sc as plsc

NODE_TYPES = ("image", "gene", "text")
TYPE_IX = {"image": 0, "gene": 1, "text": 2}
EDGE_TYPES = (
    ("image", "image", "i2i"),
    ("gene", "image", "g2i"),
    ("text", "image", "t2i"),
    ("image", "gene", "i2g"),
    ("image", "text", "i2t"),
)
HID = 256
NH = 8
DK = 32
SQRT_DK = float(DK) ** 0.5
LANES = 16
NC, NS, NW = 2, 16, 32
B_EDGE = 64


def _ceil_to(x, m):
    return (x + m - 1) // m * m


def _mm_body(x_ref, w_ref, b_ref, o_ref):
    o_ref[...] = (
        jnp.dot(x_ref[...], w_ref[...], preferred_element_type=jnp.float32)
        + b_ref[...]
    )


def _matmul_bias(x, w, b, bn=512):
    n, kdim = x.shape
    m = w.shape[1]
    npad = _ceil_to(n, bn)
    if npad != n:
        x = jnp.pad(x, ((0, npad - n), (0, 0)))
    out = pl.pallas_call(
        _mm_body,
        grid=(npad // bn,),
        in_specs=[
            pl.BlockSpec((bn, kdim), lambda i: (i, 0)),
            pl.BlockSpec((kdim, m), lambda i: (0, 0)),
            pl.BlockSpec((1, m), lambda i: (0, 0)),
        ],
        out_specs=pl.BlockSpec((bn, m), lambda i: (i, 0)),
        out_shape=jax.ShapeDtypeStruct((npad, m), jnp.float32),
    )(x, w, b.reshape(1, m))
    return out[:n]


def _make_newh_body(n_et):
    def body(*refs):
        h_ref = refs[0]
        p_refs = refs[1:1 + 6 * n_et]
        e8h = refs[1 + 6 * n_et][...]
        wa_top = refs[2 + 6 * n_et][...]
        wa_bot = refs[3 + 6 * n_et][...]
        ba = refs[4 + 6 * n_et][...]
        alpha = refs[5 + 6 * n_et][0, 0]
        o_ref = refs[6 + 6 * n_et]
        tsum_e = tsum_o = None
        for t in range(n_et):
            ae = p_refs[6 * t][...] + p_refs[6 * t + 1][...]
            ao = p_refs[6 * t + 2][...] + p_refs[6 * t + 3][...]
            dd = p_refs[6 * t + 4][...] + p_refs[6 * t + 5][...]
            denf = jnp.maximum(
                jnp.dot(dd, e8h, preferred_element_type=jnp.float32), 1e-30)
            if tsum_e is None:
                tsum_e, tsum_o = ae / denf, ao / denf
            else:
                tsum_e, tsum_o = tsum_e + ae / denf, tsum_o + ao / denf
        trans = (
            jnp.dot(tsum_e * (1.0 / n_et), wa_top,
                    preferred_element_type=jnp.float32)
            + jnp.dot(tsum_o * (1.0 / n_et), wa_bot,
                      preferred_element_type=jnp.float32)
            + ba
        )
        o_ref[...] = trans * alpha + h_ref[...] * (1.0 - alpha)
    return body


def _new_h(h, plist, e8h, wa_perm, ba, alpha, bn=512):
    n = h.shape[0]
    npad = _ceil_to(n, bn)
    n_et = len(plist)
    hp = jnp.pad(h, ((0, npad - n), (0, 0))) if npad != n else h
    args = [hp]
    in_specs = [pl.BlockSpec((bn, HID), lambda i: (i, 0))]
    for ae, ao, den in plist:
        for part, width in ((ae, 128), (ao, 128), (den, NH)):
            for ci in range(NC):
                pc = part[ci]
                if npad != n:
                    pc = jnp.pad(pc, ((0, npad - n), (0, 0)))
                args.append(pc)
                in_specs.append(pl.BlockSpec((bn, width), lambda i: (i, 0)))
    args += [e8h, wa_perm[:128], wa_perm[128:],
             ba.reshape(1, HID), alpha.reshape(1, 1)]
    in_specs += [
        pl.BlockSpec((NH, 128), lambda i: (0, 0)),
        pl.BlockSpec((128, HID), lambda i: (0, 0)),
        pl.BlockSpec((128, HID), lambda i: (0, 0)),
        pl.BlockSpec((1, HID), lambda i: (0, 0)),
        pl.BlockSpec((1, 1), lambda i: (0, 0)),
    ]
    out = pl.pallas_call(
        _make_newh_body(n_et),
        grid=(npad // bn,),
        in_specs=in_specs,
        out_specs=pl.BlockSpec((bn, HID), lambda i: (i, 0)),
        out_shape=jax.ShapeDtypeStruct((npad, HID), jnp.float32),
    )(*args)
    return out[:n]


def _head_body(h_ref, wlp_ref, blp_ref, w1_ref, b1_ref, wh_ref, bh_ref, o_ref):
    pooled = jnp.mean(h_ref[...], axis=0, keepdims=True)
    oh = jnp.dot(pooled, wlp_ref[...], preferred_element_type=jnp.float32) + blp_ref[...]
    g = jnp.dot(oh, w1_ref[...], preferred_element_type=jnp.float32) + b1_ref[...]
    o_ref[...] = jnp.dot(g, wh_ref[...], preferred_element_type=jnp.float32) + bh_ref[...]


def _head_rowbias(h_img, wlp, blp, w1, b1, wh, bh):
    n = h_img.shape[0]
    return pl.pallas_call(
        _head_body,
        in_specs=[
            pl.BlockSpec((n, HID), lambda: (0, 0)),
            pl.BlockSpec((HID, HID), lambda: (0, 0)),
            pl.BlockSpec((1, HID), lambda: (0, 0)),
            pl.BlockSpec((HID, 512), lambda: (0, 0)),
            pl.BlockSpec((1, 512), lambda: (0, 0)),
            pl.BlockSpec((512, HID), lambda: (0, 0)),
            pl.BlockSpec((1, HID), lambda: (0, 0)),
        ],
        out_specs=pl.BlockSpec((1, HID), lambda: (0, 0)),
        out_shape=jax.ShapeDtypeStruct((1, HID), jnp.float32),
    )(h_img, wlp, blp.reshape(1, HID), w1, b1.reshape(1, 512), wh, bh.reshape(1, HID))


def _edge_pass(ke, ko, qe, qo, ve, vo, src, dst, easc, nd):
    e = src.shape[0]
    e_pad = _ceil_to(e, NW * B_EDGE)
    pad = e_pad - e
    if pad:
        src = jnp.concatenate([src, jnp.zeros((pad,), jnp.int32)])
        dst = jnp.concatenate([dst, jnp.full((pad,), nd, jnp.int32)])
        easc = jnp.concatenate([easc, jnp.zeros((pad,), jnp.float32)])
    dstrow = lax.shift_right_logical(dst, 4)
    packed = jnp.stack(
        [src.reshape(-1, B_EDGE), dst.reshape(-1, B_EDGE),
         dstrow.reshape(-1, B_EDGE),
         lax.bitcast_convert_type(easc, jnp.int32).reshape(-1, B_EDGE)],
        axis=1)
    nd_pad = _ceil_to(nd + 1, 2048)
    nrows_den = nd_pad // 16
    epw = e_pad // NW
    nchunks = epw // B_EDGE
    rows_z = nd_pad // NS
    rows_dz = nrows_den // NS
    nhalf = 128

    def body(ke_hbm, ko_hbm, qe_hbm, qo_hbm, ve_hbm, vo_hbm,
             packed_hbm, z_hbm,
             agge_hbm, aggo_hbm, den_hbm,
             pbuf, b1, b2, accbuf, exbuf, denrow,
             agge_s, aggo_s, den_s, sem1, sem2, sem3):
        c = lax.axis_index("c")
        s = lax.axis_index("s")
        wid = s * NC + c
        zero16 = jnp.zeros((LANES,), jnp.float32)
        iota16 = lax.iota(jnp.int32, LANES)
        perm = iota16 ^ 8
        lmask = iota16 < NH
        iota8m = iota16 & 7
        zr0 = s * rows_z
        dzr0 = s * rows_dz
        pltpu.sync_copy(z_hbm.at[pl.ds(zr0, rows_z)],
                        agge_s.at[pl.ds(zr0, rows_z)])
        pltpu.sync_copy(z_hbm.at[pl.ds(zr0, rows_z)],
                        aggo_s.at[pl.ds(zr0, rows_z)])
        pltpu.sync_copy(z_hbm.at[pl.ds(dzr0, rows_dz)],
                        den_s.at[pl.ds(dzr0, rows_dz)])
        plsc.subcore_barrier()

        ebase = wid * epw
        ngroups = B_EDGE // LANES

        def chunk(i, _):
            gci = wid * nchunks + i
            pltpu.sync_copy(packed_hbm.at[gci], pbuf)
            src_ix = pbuf.at[0]
            dst_ix = pbuf.at[1]
            dstrow_ix = pbuf.at[2]

            cp1 = pltpu.async_copy(ke_hbm.at[src_ix], b1, sem1)
            cp2 = pltpu.async_copy(qe_hbm.at[dst_ix], b2, sem2)
            cp1.wait()
            cp2.wait()

            def group1(g, _):
                for j in range(LANES):
                    ej = g * LANES + j
                    acc = zero16
                    for r in range(nhalf // LANES):
                        acc = acc + (b1[ej, pl.ds(r * LANES, LANES)]
                                     * b2[ej, pl.ds(r * LANES, LANES)])
                    accbuf[pl.ds(ej * LANES, LANES)] = acc
                return 0

            lax.fori_loop(0, ngroups, group1, 0)

            cp1 = pltpu.async_copy(ko_hbm.at[src_ix], b1, sem1)
            cp2 = pltpu.async_copy(qo_hbm.at[dst_ix], b2, sem2)
            cp1.wait()
            cp2.wait()

            def group2(g, _):
                e0 = g * LANES
                dst16 = pbuf[1, pl.ds(e0, LANES)]
                easc16 = plsc.bitcast(pbuf[3, pl.ds(e0, LANES)],
                                      jnp.float32)
                for j in range(LANES):
                    ej = e0 + j
                    acc = accbuf[pl.ds(ej * LANES, LANES)]
                    for r in range(nhalf // LANES):
                        acc = acc + (b1[ej, pl.ds(r * LANES, LANES)]
                                     * b2[ej, pl.ds(r * LANES, LANES)])
                    accs = lax.gather(
                        acc, perm[:, None],
                        lax.GatherDimensionNumbers(
                            offset_dims=(), collapsed_slice_dims=(0,),
                            start_index_map=(0,)),
                        slice_sizes=(1,),
                        mode=lax.GatherScatterMode.PROMISE_IN_BOUNDS)
                    ex = jnp.exp((acc + accs)
                                 * jnp.full((LANES,), easc16[j]))
                    exbuf[pl.ds(ej * LANES, LANES)] = ex
                    for q in range(nhalf // LANES):
                        denrow[ej, pl.ds(q * LANES, LANES)] = zero16
                    col = (dst16[j] & 15) * NH + iota8m
                    plsc.store_scatter(
                        denrow, [jnp.full((LANES,), ej, jnp.int32), col],
                        ex, mask=lmask)
                return 0

            lax.fori_loop(0, ngroups, group2, 0)

            cp1 = pltpu.async_copy(ve_hbm.at[src_ix], b1, sem1)
            cp2 = pltpu.async_copy(vo_hbm.at[src_ix], b2, sem2)
            cp1.wait()

            def group3(g, _):
                for j in range(LANES):
                    ej = g * LANES + j
                    ex = exbuf[pl.ds(ej * LANES, LANES)]
                    for r in range(nhalf // LANES):
                        b1[ej, pl.ds(r * LANES, LANES)] = (
                            b1[ej, pl.ds(r * LANES, LANES)] * ex)
                return 0

            lax.fori_loop(0, ngroups, group3, 0)
            cs1 = pltpu.async_copy(b1, agge_s.at[dst_ix], sem1, add=True)
            cp2.wait()

            def group4(g, _):
                for j in range(LANES):
                    ej = g * LANES + j
                    ex = exbuf[pl.ds(ej * LANES, LANES)]
                    for r in range(nhalf // LANES):
                        b2[ej, pl.ds(r * LANES, LANES)] = (
                            b2[ej, pl.ds(r * LANES, LANES)] * ex)
                return 0

            lax.fori_loop(0, ngroups, group4, 0)
            cs2 = pltpu.async_copy(b2, aggo_s.at[dst_ix], sem2, add=True)
            cs3 = pltpu.async_copy(denrow, den_s.at[dstrow_ix], sem3,
                                   add=True)
            cs1.wait()
            cs2.wait()
            cs3.wait()
            return 0

        lax.fori_loop(0, nchunks, chunk, 0)
        plsc.subcore_barrier()
        pltpu.sync_copy(agge_s.at[pl.ds(zr0, rows_z)],
                        agge_hbm.at[c, pl.ds(zr0, rows_z)])
        pltpu.sync_copy(aggo_s.at[pl.ds(zr0, rows_z)],
                        aggo_hbm.at[c, pl.ds(zr0, rows_z)])
        pltpu.sync_copy(den_s.at[pl.ds(dzr0, rows_dz)],
                        den_hbm.at[c, pl.ds(dzr0, rows_dz)])

    mesh = plsc.VectorSubcoreMesh(core_axis_name="c", subcore_axis_name="s")
    f = pl.kernel(
        body,
        mesh=mesh,
        compiler_params=pltpu.CompilerParams(needs_layout_passes=False),
        out_type=(
            jax.ShapeDtypeStruct((NC, nd_pad, 128), jnp.float32),
            jax.ShapeDtypeStruct((NC, nd_pad, 128), jnp.float32),
            jax.ShapeDtypeStruct((NC, nrows_den, 128), jnp.float32),
        ),
        scratch_types=[
            pltpu.VMEM((4, B_EDGE), jnp.int32),
            pltpu.VMEM((B_EDGE, 128), jnp.float32),
            pltpu.VMEM((B_EDGE, 128), jnp.float32),
            pltpu.VMEM((B_EDGE * LANES,), jnp.float32),
            pltpu.VMEM((B_EDGE * LANES,), jnp.float32),
            pltpu.VMEM((B_EDGE, 128), jnp.float32),
            pltpu.VMEM_SHARED((nd_pad, 128), jnp.float32),
            pltpu.VMEM_SHARED((nd_pad, 128), jnp.float32),
            pltpu.VMEM_SHARED((nrows_den, 128), jnp.float32),
            pltpu.SemaphoreType.DMA,
            pltpu.SemaphoreType.DMA,
            pltpu.SemaphoreType.DMA,
        ],
    )
    zeros = jnp.zeros((nd_pad, 128), jnp.float32)
    agg_e, agg_o, den = f(ke, ko, qe, qo, ve, vo, packed, zeros)
    den = den.reshape(NC, nd_pad, NH)
    return agg_e[:, :nd], agg_o[:, :nd], den[:, :nd]


def _perm_cols(w):
    return w.reshape(w.shape[:-1] + (NH, DK)).swapaxes(-2, -1).reshape(w.shape)


def _perm_rows(w):
    return (w.reshape((NH, DK) + w.shape[-1:])
            .swapaxes(0, 1).reshape((NH * DK,) + w.shape[-1:]))


def kernel(feat_image, feat_gene, feat_text, src_i2i, dst_i2i, sim_i2i,
           src_g2i, dst_g2i, sim_g2i, src_t2i, dst_t2i, sim_t2i,
           src_i2g, dst_i2g, sim_i2g, src_i2t, dst_i2t, sim_i2t,
           Wad, bad, Wk, bk, Wq, bq, Wv, bv, Wa, ba, ew, eb, skip,
           Wlp, blp, Wattn, Whead1, bhead1, Whead, bhead):
    feats = {"image": feat_image, "gene": feat_gene, "text": feat_text}
    srcs = {"i2i": src_i2i, "g2i": src_g2i, "t2i": src_t2i,
            "i2g": src_i2g, "i2t": src_i2t}
    dsts = {"i2i": dst_i2i, "g2i": dst_g2i, "t2i": dst_t2i,
            "i2g": dst_i2g, "i2t": dst_i2t}
    sims = {"i2i": sim_i2i, "g2i": sim_g2i, "t2i": sim_t2i,
            "i2g": sim_i2g, "i2t": sim_i2t}
    nnodes = {nt: feats[nt].shape[0] for nt in NODE_TYPES}

    li = jnp.arange(NH)[:, None]
    ji = jnp.arange(128)[None, :]
    e8h = (li == (ji % NH)).astype(jnp.float32)

    h = {nt: _matmul_bias(feats[nt], Wad[TYPE_IX[nt]], bad[TYPE_IX[nt]])
         for nt in NODE_TYPES}

    for l in range(2):
        etypes_l = [et for et in EDGE_TYPES if l == 0 or et[1] == "image"]
        src_types = {st for st, _, _ in etypes_l}
        dst_types = {dt for _, dt, _ in etypes_l}
        kqv = {}
        for nt in NODE_TYPES:
            if nt not in src_types and nt not in dst_types:
                continue
            i = TYPE_IX[nt]
            w3 = jnp.concatenate(
                [_perm_cols(Wk[l, i]), _perm_cols(Wq[l, i]),
                 _perm_cols(Wv[l, i])], axis=1)
            b3 = jnp.concatenate(
                [_perm_cols(bk[l, i]), _perm_cols(bq[l, i]),
                 _perm_cols(bv[l, i])], axis=0)
            proj = _matmul_bias(h[nt], w3, b3)
            kqv[nt] = tuple(proj[:, 128 * t:128 * (t + 1)] for t in range(6))
        partials = {}
        for st, dt, en in etypes_l:
            easc = (sims[en] * ew[l] + eb[l]) * (1.0 / SQRT_DK)
            partials[en] = _edge_pass(
                kqv[st][0], kqv[st][1], kqv[dt][2], kqv[dt][3],
                kqv[st][4], kqv[st][5],
                srcs[en], dsts[en], easc, nnodes[dt])
        new_h = dict(h)
        for dt in NODE_TYPES:
            plist = [partials[en] for st, dtt, en in etypes_l if dtt == dt]
            if not plist:
                continue
            i = TYPE_IX[dt]
            alpha = jax.nn.sigmoid(skip[l, i])
            new_h[dt] = _new_h(h[dt], plist, e8h, _perm_rows(Wa[l, i]),
                               ba[l, i], alpha)
        h = new_h

    rowbias = _head_rowbias(h["image"], Wlp[0], blp[0],
                            Whead1, bhead1, Whead, bhead)
    return _matmul_bias(feat_image, Whead, rowbias)
